# baseline (speedup 1.0000x reference)
import jax
import jax.numpy as jnp
from jax import lax
from jax.experimental import pallas as pl
from jax.experimental.pallas import tpu as pltpu
from jax.experimental.pallas import tpu_sc as plsc

XK = 1000000
NBATCH = 16384
SMOOTH_EPS = 1e-6

NC = 2
NS = 16
NW = NC * NS
BPW = NBATCH // NW
ROWS_PER_W = BPW // 128


def _stats_body(emb_ref, out_ref):
    x = emb_ref[...]
    m = jnp.max(x)
    s = jnp.sum(jnp.exp(x - m))
    scale = (1.0 - SMOOTH_EPS) / s
    lane = lax.broadcasted_iota(jnp.int32, (32,), 0)
    out_ref[...] = jnp.where(lane < 16, m, scale)


def _gather_body(emb_hbm, idx_hbm, stats_hbm, out_hbm, idx_v, rows_v, stats_v, sem):
    wid = lax.axis_index("s") * NC + lax.axis_index("c")
    base = wid * ROWS_PER_W
    pltpu.sync_copy(idx_hbm.at[pl.ds(base, ROWS_PER_W)], idx_v)
    pltpu.sync_copy(stats_hbm, stats_v)
    copies = [
        pltpu.async_copy(emb_hbm.at[idx_v.at[j]], rows_v.at[j], sem)
        for j in range(ROWS_PER_W)
    ]
    for c in copies:
        c.wait()
    m = stats_v[pl.ds(0, 16)]
    scale = stats_v[pl.ds(16, 16)]
    epsk = jnp.float32(SMOOTH_EPS / XK)
    for j in range(ROWS_PER_W):
        for l in range(8):
            v = rows_v[j, pl.ds(l * 16, 16)]
            rows_v[j, pl.ds(l * 16, 16)] = jnp.exp(v - m) * scale + epsk
    pltpu.sync_copy(rows_v, out_hbm.at[pl.ds(base, ROWS_PER_W)])


@jax.jit
def kernel(inputs, embedding):
    stats = pl.pallas_call(
        _stats_body,
        out_shape=jax.ShapeDtypeStruct((32,), jnp.float32),
    )(embedding)

    idx = inputs.reshape(NBATCH // 128, 128).astype(jnp.int32)

    mesh = plsc.VectorSubcoreMesh(core_axis_name="c", subcore_axis_name="s")
    gathered = pl.kernel(
        _gather_body,
        mesh=mesh,
        out_type=jax.ShapeDtypeStruct((NBATCH // 128, 128), jnp.float32),
        scratch_types=[
            pltpu.VMEM((ROWS_PER_W, 128), jnp.int32),
            pltpu.VMEM((ROWS_PER_W, 128), jnp.float32),
            pltpu.VMEM((32,), jnp.float32),
            pltpu.SemaphoreType.DMA,
        ],
    )(embedding, idx, stats)

    return gathered.reshape(NBATCH, 1)


# speedup vs baseline: 1.0999x; 1.0999x over previous
import jax
import jax.numpy as jnp
from jax import lax
from jax.experimental import pallas as pl
from jax.experimental.pallas import tpu as pltpu
from jax.experimental.pallas import tpu_sc as plsc

XK = 1000000
NBATCH = 16384
SMOOTH_EPS = 1e-6

NC = 2
NS = 16
NW = NC * NS
BPW = NBATCH // NW
ROWS_PER_W = BPW // 128


def _stats_body(emb_ref, out_ref):
    x = emb_ref[...]
    m = jnp.max(x)
    s = jnp.sum(jnp.exp(x - m))
    scale = (1.0 - SMOOTH_EPS) / s
    row = lax.broadcasted_iota(jnp.int32, (8, 128), 0)
    out_ref[...] = jnp.where(row < 1, m, scale)


def _gather_body(emb_hbm, idx_hbm, out_hbm, idx_v, rows_v, sem):
    wid = lax.axis_index("s") * NC + lax.axis_index("c")
    base = wid * ROWS_PER_W
    pltpu.sync_copy(idx_hbm.at[pl.ds(base, ROWS_PER_W)], idx_v)
    copies = [
        pltpu.async_copy(emb_hbm.at[idx_v.at[j]], rows_v.at[j], sem)
        for j in range(ROWS_PER_W)
    ]
    for c in copies:
        c.wait()
    pltpu.sync_copy(rows_v, out_hbm.at[pl.ds(base, ROWS_PER_W)])


def _apply_body(g_ref, stats_ref, out_ref):
    s = stats_ref[...]
    m = s[0, 0]
    scale = s[1, 0]
    g = g_ref[...]
    out_ref[...] = jnp.exp(g - m) * scale + jnp.float32(SMOOTH_EPS / XK)


@jax.jit
def kernel(inputs, embedding):
    idx = inputs.reshape(NBATCH // 128, 128).astype(jnp.int32)

    mesh = plsc.VectorSubcoreMesh(core_axis_name="c", subcore_axis_name="s")
    gathered = pl.kernel(
        _gather_body,
        mesh=mesh,
        out_type=jax.ShapeDtypeStruct((NBATCH // 128, 128), jnp.float32),
        scratch_types=[
            pltpu.VMEM((ROWS_PER_W, 128), jnp.int32),
            pltpu.VMEM((ROWS_PER_W, 128), jnp.float32),
            pltpu.SemaphoreType.DMA,
        ],
    )(embedding, idx)

    stats = pl.pallas_call(
        _stats_body,
        out_shape=jax.ShapeDtypeStruct((8, 128), jnp.float32),
    )(embedding)

    out = pl.pallas_call(
        _apply_body,
        out_shape=jax.ShapeDtypeStruct((NBATCH // 128, 128), jnp.float32),
    )(gathered, stats)

    return out.reshape(NBATCH, 1)


# speedup vs baseline: 1.1340x; 1.0311x over previous
import jax
import jax.numpy as jnp
from jax import lax
from jax.experimental import pallas as pl
from jax.experimental.pallas import tpu as pltpu
from jax.experimental.pallas import tpu_sc as plsc

XK = 1000000
NBATCH = 16384
SMOOTH_EPS = 1e-6

NC = 2
NS = 16
NW = NC * NS
BPW = NBATCH // NW
ROWS_PER_W = BPW // 128


def _stats_body(emb_ref, out_ref):
    x = emb_ref[...]
    m = jnp.max(x)
    s = jnp.sum(jnp.exp(x - m))
    scale = (1.0 - SMOOTH_EPS) / s
    row = lax.broadcasted_iota(jnp.int32, (8, 128), 0)
    out_ref[...] = jnp.where(row < 1, m, scale)


def _gather_body(emb_hbm, idx_hbm, out_hbm, idx_v, rows_v, sem):
    wid = lax.axis_index("s") * NC + lax.axis_index("c")
    base = wid * ROWS_PER_W
    pltpu.sync_copy(idx_hbm.at[pl.ds(base, ROWS_PER_W)], idx_v)
    copies = [
        pltpu.async_copy(emb_hbm.at[idx_v.at[j]], rows_v.at[j], sem)
        for j in range(ROWS_PER_W)
    ]
    for c in copies:
        c.wait()
    pltpu.sync_copy(rows_v, out_hbm.at[pl.ds(base, ROWS_PER_W)])


def _apply_body(g_ref, stats_ref, out_ref):
    s = stats_ref[...]
    m = s[0, 0]
    scale = s[1, 0]
    g = g_ref[...]
    out_ref[...] = jnp.exp(g - m) * scale + jnp.float32(SMOOTH_EPS / XK)


@jax.jit
def kernel(inputs, embedding):
    idx = inputs.reshape(NBATCH // 128, 128).astype(jnp.int32)

    mesh = plsc.VectorSubcoreMesh(core_axis_name="c", subcore_axis_name="s")
    gathered = pl.kernel(
        _gather_body,
        mesh=mesh,
        out_type=jax.ShapeDtypeStruct((NBATCH // 128, 128), jnp.float32),
        scratch_types=[
            pltpu.VMEM((ROWS_PER_W, 128), jnp.int32),
            pltpu.VMEM((ROWS_PER_W, 128), jnp.float32),
            pltpu.SemaphoreType.DMA,
        ],
    )(embedding, idx)

    stats = pl.pallas_call(
        _stats_body,
        out_shape=jax.ShapeDtypeStruct((8, 128), jnp.float32),
    )(embedding.reshape(1000, 1000))

    out = pl.pallas_call(
        _apply_body,
        out_shape=jax.ShapeDtypeStruct((NBATCH // 128, 128), jnp.float32),
    )(gathered, stats)

    return out.reshape(NBATCH, 1)
